# baseline (speedup 1.0000x reference)
import functools

import jax
import jax.numpy as jnp
from jax import lax
from jax.experimental import pallas as pl
from jax.experimental.pallas import tpu as pltpu
from jax.experimental.pallas import tpu_sc as plsc


def kernel(x, continuous_x_res, tables):
    B, C = x.shape
    _, NCONT, D = continuous_x_res.shape
    V = tables.shape[1]
    OUT_C = C + NCONT
    CH = C // 2

    xT = x.T
    tab_a = tables[:CH]
    tab_b = tables[CH:]

    NC, NS = 2, 16
    NW = NC * NS
    b_per_w = B // NW
    BK = 64
    steps = b_per_w // BK

    mesh = plsc.VectorSubcoreMesh(core_axis_name="c", subcore_axis_name="s")

    @functools.partial(
        pl.kernel,
        mesh=mesh,
        out_type=jax.ShapeDtypeStruct((B, OUT_C, D), jnp.float32),
        compiler_params=pltpu.CompilerParams(use_tc_tiling_on_sc=False),
        scratch_types=[
            pltpu.VMEM((C, BK), jnp.int32),
            pltpu.VMEM((C, BK, D), jnp.float32),
            pltpu.SemaphoreType.DMA,
            pltpu.SemaphoreType.DMA,
        ],
    )
    def k(taba_hbm, tabb_hbm, idx_hbm, cont_hbm, out_hbm, idx_v, gbuf,
          sem_g, sem_c):
        wid = lax.axis_index("s") * NC + lax.axis_index("c")
        base = wid * b_per_w

        cont_cp = pltpu.async_copy(
            cont_hbm.at[pl.ds(base, b_per_w)],
            out_hbm.at[pl.ds(base, b_per_w), pl.ds(C, NCONT)],
            sem_c,
        )

        @pl.loop(0, steps)
        def _(t):
            row0 = base + t * BK
            pltpu.sync_copy(idx_hbm.at[:, pl.ds(row0, BK)], idx_v)
            gathers = []
            for c in range(C):
                src = taba_hbm.at[c] if c < CH else tabb_hbm.at[c - CH]
                gathers.append(pltpu.async_copy(
                    src.at[idx_v.at[c]],
                    gbuf.at[c],
                    sem_g,
                ))
            for cp in gathers:
                cp.wait()
            writes = []
            for c in range(C):
                writes.append(pltpu.async_copy(
                    gbuf.at[c],
                    out_hbm.at[pl.ds(row0, BK), c],
                    sem_g,
                ))
            for cp in writes:
                cp.wait()

        cont_cp.wait()

    return k(tab_a, tab_b, xT, continuous_x_res)


# speedup vs baseline: 2.1146x; 2.1146x over previous
import functools

import jax
import jax.numpy as jnp
from jax import lax
from jax.experimental import pallas as pl
from jax.experimental.pallas import tpu as pltpu
from jax.experimental.pallas import tpu_sc as plsc


def kernel(x, continuous_x_res, tables):
    B, C = x.shape
    _, NCONT, D = continuous_x_res.shape
    V = tables.shape[1]
    OUT_C = C + NCONT
    R = C * D

    tabT = tables.transpose(0, 2, 1).reshape(R, V)
    xT = x.T
    contT = continuous_x_res.transpose(1, 2, 0).reshape(NCONT * D, B)

    NC, NS = 2, 16
    NW = NC * NS
    rpw = R // NW
    crw = (NCONT * D) // NW

    mesh = plsc.VectorSubcoreMesh(core_axis_name="c", subcore_axis_name="s")

    @functools.partial(
        pl.kernel,
        mesh=mesh,
        out_type=jax.ShapeDtypeStruct((OUT_C * D, B), jnp.float32),
        compiler_params=pltpu.CompilerParams(use_tc_tiling_on_sc=False,
                                             needs_layout_passes=False),
        scratch_types=[
            pltpu.VMEM((V,), jnp.float32),
            pltpu.VMEM((B,), jnp.int32),
            pltpu.VMEM((B,), jnp.float32),
            pltpu.SemaphoreType.DMA,
            pltpu.SemaphoreType.DMA,
        ],
    )
    def k(tab_hbm, idx_hbm, cont_hbm, out_hbm, row_v, idx_v, out_v,
          sem_r, sem_c):
        wid = lax.axis_index("s") * NC + lax.axis_index("c")

        cont_cp = pltpu.async_copy(
            cont_hbm.at[pl.ds(wid * crw, crw)],
            out_hbm.at[pl.ds(R + wid * crw, crw)],
            sem_c,
        )

        @pl.loop(0, rpw)
        def _(i):
            r = wid * rpw + i
            c = r // D
            pltpu.sync_copy(idx_hbm.at[c], idx_v)
            pltpu.sync_copy(tab_hbm.at[r], row_v)

            @pl.loop(0, B, step=16)
            def _(b0):
                idx16 = idx_v[pl.ds(b0, 16)]
                out_v[pl.ds(b0, 16)] = plsc.load_gather(row_v, [idx16])

            pltpu.sync_copy(out_v, out_hbm.at[r])

        cont_cp.wait()

    out2 = k(tabT, xT, contT)
    return out2.reshape(OUT_C, D, B).transpose(2, 0, 1)
